# baseline (speedup 1.0000x reference)
import jax
import jax.numpy as jnp
from jax.experimental import pallas as pl
from jax.experimental.pallas import tpu as pltpu

EPS = 1e-5


def _fc_relu_body(x_ref, w_ref, b_ref, o_ref):
    x = x_ref[...].astype(jnp.bfloat16)
    w = w_ref[...].astype(jnp.bfloat16)
    acc = jnp.dot(x, w, preferred_element_type=jnp.float32)
    o_ref[...] = jnp.maximum(acc + b_ref[...], 0.0)


def _fc_relu(x, w, b, block_n):
    m, k = x.shape
    _, n = w.shape
    grid = (n // block_n,)
    return pl.pallas_call(
        _fc_relu_body,
        grid=grid,
        in_specs=[
            pl.BlockSpec((m, k), lambda i: (0, 0)),
            pl.BlockSpec((k, block_n), lambda i: (0, i)),
            pl.BlockSpec((1, block_n), lambda i: (0, i)),
        ],
        out_specs=pl.BlockSpec((m, block_n), lambda i: (0, i)),
        out_shape=jax.ShapeDtypeStruct((m, n), jnp.float32),
    )(x, w, b.reshape(1, n))


def _moe_body(h_ref, gw_ref, gb_ref, ebias_ref, ew_ref, eb_ref, o_ref):
    e = pl.program_id(1)
    h = h_ref[...]
    hb = h.astype(jnp.bfloat16)
    scores = jnp.dot(hb, gw_ref[...].astype(jnp.bfloat16),
                     preferred_element_type=jnp.float32)
    scores = (scores + gb_ref[...]) + ebias_ref[...]
    chosen = jnp.argmax(scores, axis=1)
    mask = (chosen == e)[:, None]
    hm = jnp.where(mask, hb, jnp.bfloat16(0.0))
    contrib = jnp.dot(hm, ew_ref[0].astype(jnp.bfloat16),
                      preferred_element_type=jnp.float32)
    contrib = contrib + jnp.where(mask, eb_ref[0], 0.0)

    @pl.when(e == 0)
    def _init():
        o_ref[...] = contrib

    @pl.when(e > 0)
    def _acc():
        o_ref[...] += contrib


def _moe_head(h, gw, gb, ebias, ew, eb, block_c):
    b_sz, hdim = h.shape
    n_experts, _, cdim = ew.shape
    grid = (pl.cdiv(cdim, block_c), n_experts)
    return pl.pallas_call(
        _moe_body,
        grid=grid,
        in_specs=[
            pl.BlockSpec((b_sz, hdim), lambda i, j: (0, 0)),
            pl.BlockSpec((hdim, n_experts), lambda i, j: (0, 0)),
            pl.BlockSpec((1, n_experts), lambda i, j: (0, 0)),
            pl.BlockSpec((1, n_experts), lambda i, j: (0, 0)),
            pl.BlockSpec((1, hdim, block_c), lambda i, j: (j, 0, i)),
            pl.BlockSpec((1, 1, block_c), lambda i, j: (j, 0, i)),
        ],
        out_specs=pl.BlockSpec((b_sz, block_c), lambda i, j: (0, i)),
        out_shape=jax.ShapeDtypeStruct((b_sz, cdim), jnp.float32),
    )(h, gw, gb.reshape(1, n_experts), ebias.reshape(1, n_experts),
      ew, eb.reshape(n_experts, 1, cdim))


def _conv(x, w, b, stride, pad):
    out = jax.lax.conv_general_dilated(
        x, w, (stride, stride), [(pad, pad), (pad, pad)],
        dimension_numbers=('NCHW', 'OIHW', 'NCHW'))
    return out + b[None, :, None, None]


def _bn(x, g, b):
    scale = g / jnp.sqrt(1.0 + EPS)
    return x * scale[None, :, None, None] + b[None, :, None, None]


def _maxpool(x):
    return jax.lax.reduce_window(x, -jnp.inf, jax.lax.max,
                                 (1, 1, 3, 3), (1, 1, 2, 2), 'VALID')


def kernel(x, params, expert_bias):
    p = params
    t = _maxpool(jax.nn.relu(_bn(_conv(x, p['c1w'], p['c1b'], 4, 0), p['bn1g'], p['bn1b'])))
    t = _maxpool(jax.nn.relu(_bn(_conv(t, p['c2w'], p['c2b'], 1, 2), p['bn2g'], p['bn2b'])))
    t = jax.nn.relu(_bn(_conv(t, p['c3w'], p['c3b'], 1, 1), p['bn3g'], p['bn3b']))
    t = jax.nn.relu(_bn(_conv(t, p['c4w'], p['c4b'], 1, 1), p['bn4g'], p['bn4b']))
    t = _maxpool(jax.nn.relu(_bn(_conv(t, p['c5w'], p['c5b'], 1, 1), p['bn5g'], p['bn5b'])))
    h0 = t.reshape(t.shape[0], -1)

    h1 = _fc_relu(h0, p['fc1w'], p['fc1b'], block_n=512)
    h2 = _fc_relu(h1, p['fc2w'], p['fc2b'], block_n=512)

    out = _moe_head(h2, p['gw'], p['gb'], expert_bias, p['ew'], p['eb'],
                    block_c=1000)
    return out


# speedup vs baseline: 1.1144x; 1.1144x over previous
import jax
import jax.numpy as jnp
from jax.experimental import pallas as pl
from jax.experimental.pallas import tpu as pltpu

EPS = 1e-5


def _fc_relu_body(x_ref, w_ref, b_ref, o_ref):
    x = x_ref[...].astype(jnp.bfloat16)
    w = w_ref[...].astype(jnp.bfloat16)
    acc = jnp.dot(x, w, preferred_element_type=jnp.float32)
    o_ref[...] = jnp.maximum(acc + b_ref[...], 0.0)


def _fc_relu(x, w, b, block_n):
    m, k = x.shape
    _, n = w.shape
    grid = (n // block_n,)
    return pl.pallas_call(
        _fc_relu_body,
        grid=grid,
        in_specs=[
            pl.BlockSpec((m, k), lambda i: (0, 0)),
            pl.BlockSpec((k, block_n), lambda i: (0, i)),
            pl.BlockSpec((1, block_n), lambda i: (0, i)),
        ],
        out_specs=pl.BlockSpec((m, block_n), lambda i: (0, i)),
        out_shape=jax.ShapeDtypeStruct((m, n), jnp.float32),
    )(x, w, b.reshape(1, n))


def _moe_body(h_ref, gw_ref, gb_ref, ebias_ref, ew_ref, eb_ref, o_ref):
    e = pl.program_id(1)
    hb = h_ref[...].astype(jnp.bfloat16)
    scores = jnp.dot(hb, gw_ref[...].astype(jnp.bfloat16),
                     preferred_element_type=jnp.float32)
    scores = (scores + gb_ref[...]) + ebias_ref[...]
    chosen = jnp.argmax(scores, axis=1)
    mask = (chosen == e)[:, None]
    hm = jnp.where(mask, hb, jnp.bfloat16(0.0))
    contrib = jnp.dot(hm, ew_ref[...].astype(jnp.bfloat16),
                      preferred_element_type=jnp.float32)
    contrib = contrib + jnp.where(mask, eb_ref[0], 0.0)

    @pl.when(e == 0)
    def _init():
        o_ref[...] = contrib

    @pl.when(e > 0)
    def _acc():
        o_ref[...] += contrib


def _moe_head(h, gw, gb, ebias, ew, eb, block_c):
    b_sz, hdim = h.shape
    n_experts, _, cdim = ew.shape
    grid = (pl.cdiv(cdim, block_c), n_experts)
    return pl.pallas_call(
        _moe_body,
        grid=grid,
        in_specs=[
            pl.BlockSpec((b_sz, hdim), lambda i, j: (0, 0)),
            pl.BlockSpec((hdim, n_experts), lambda i, j: (0, 0)),
            pl.BlockSpec((1, n_experts), lambda i, j: (0, 0)),
            pl.BlockSpec((1, n_experts), lambda i, j: (0, 0)),
            pl.BlockSpec((hdim, block_c), lambda i, j: (j, i)),
            pl.BlockSpec((1, 1, block_c), lambda i, j: (j, 0, i)),
        ],
        out_specs=pl.BlockSpec((b_sz, block_c), lambda i, j: (0, i)),
        out_shape=jax.ShapeDtypeStruct((b_sz, cdim), jnp.float32),
    )(h, gw, gb.reshape(1, n_experts), ebias.reshape(1, n_experts),
      ew.reshape(n_experts * hdim, cdim), eb.reshape(n_experts, 1, cdim))


def _conv(x, w, b, stride, pad):
    out = jax.lax.conv_general_dilated(
        x, w, (stride, stride), [(pad, pad), (pad, pad)],
        dimension_numbers=('NCHW', 'OIHW', 'NCHW'))
    return out + b[None, :, None, None]


def _bn(x, g, b):
    scale = g / jnp.sqrt(1.0 + EPS)
    return x * scale[None, :, None, None] + b[None, :, None, None]


def _maxpool(x):
    return jax.lax.reduce_window(x, -jnp.inf, jax.lax.max,
                                 (1, 1, 3, 3), (1, 1, 2, 2), 'VALID')


def kernel(x, params, expert_bias):
    p = params
    t = _maxpool(jax.nn.relu(_bn(_conv(x, p['c1w'], p['c1b'], 4, 0), p['bn1g'], p['bn1b'])))
    t = _maxpool(jax.nn.relu(_bn(_conv(t, p['c2w'], p['c2b'], 1, 2), p['bn2g'], p['bn2b'])))
    t = jax.nn.relu(_bn(_conv(t, p['c3w'], p['c3b'], 1, 1), p['bn3g'], p['bn3b']))
    t = jax.nn.relu(_bn(_conv(t, p['c4w'], p['c4b'], 1, 1), p['bn4g'], p['bn4b']))
    t = _maxpool(jax.nn.relu(_bn(_conv(t, p['c5w'], p['c5b'], 1, 1), p['bn5g'], p['bn5b'])))
    h0 = t.reshape(t.shape[0], -1)

    h1 = _fc_relu(h0, p['fc1w'], p['fc1b'], block_n=512)
    h2 = _fc_relu(h1, p['fc2w'], p['fc2b'], block_n=512)

    out = _moe_head(h2, p['gw'], p['gb'], expert_bias, p['ew'], p['eb'],
                    block_c=1000)
    return out


# speedup vs baseline: 1.1183x; 1.0035x over previous
import jax
import jax.numpy as jnp
from jax.experimental import pallas as pl
from jax.experimental.pallas import tpu as pltpu

EPS = 1e-5


def _fc_relu_body(x_ref, w_ref, b_ref, o_ref):
    x = x_ref[...].astype(jnp.bfloat16)
    w = w_ref[...].astype(jnp.bfloat16)
    acc = jnp.dot(x, w, preferred_element_type=jnp.float32)
    o_ref[...] = jnp.maximum(acc + b_ref[...], 0.0)


def _fc_relu(x, w, b, block_n):
    m, k = x.shape
    _, n = w.shape
    grid = (n // block_n,)
    return pl.pallas_call(
        _fc_relu_body,
        grid=grid,
        in_specs=[
            pl.BlockSpec((m, k), lambda i: (0, 0)),
            pl.BlockSpec((k, block_n), lambda i: (0, i)),
            pl.BlockSpec((1, block_n), lambda i: (0, i)),
        ],
        out_specs=pl.BlockSpec((m, block_n), lambda i: (0, i)),
        out_shape=jax.ShapeDtypeStruct((m, n), jnp.float32),
    )(x, w, b.reshape(1, n))


def _moe_body(h_ref, gw_ref, gb_ref, ebias_ref, ew_ref, eb_ref, o_ref):
    e = pl.program_id(1)
    hb = h_ref[...].astype(jnp.bfloat16)
    scores = jnp.dot(hb, gw_ref[...].astype(jnp.bfloat16),
                     preferred_element_type=jnp.float32)
    scores = (scores + gb_ref[...]) + ebias_ref[...]
    chosen = jnp.argmax(scores, axis=1)
    mask = (chosen == e)[:, None]
    hm = jnp.where(mask, hb, jnp.bfloat16(0.0))
    contrib = jnp.dot(hm, ew_ref[...].astype(jnp.bfloat16),
                      preferred_element_type=jnp.float32)
    contrib = contrib + jnp.where(mask, eb_ref[0], 0.0)

    @pl.when(e == 0)
    def _init():
        o_ref[...] = contrib

    @pl.when(e > 0)
    def _acc():
        o_ref[...] += contrib


def _moe_head(h, gw, gb, ebias, ew, eb, block_c):
    b_sz, hdim = h.shape
    n_experts, _, cdim = ew.shape
    grid = (pl.cdiv(cdim, block_c), n_experts)
    return pl.pallas_call(
        _moe_body,
        grid=grid,
        in_specs=[
            pl.BlockSpec((b_sz, hdim), lambda i, j: (0, 0)),
            pl.BlockSpec((hdim, n_experts), lambda i, j: (0, 0)),
            pl.BlockSpec((1, n_experts), lambda i, j: (0, 0)),
            pl.BlockSpec((1, n_experts), lambda i, j: (0, 0)),
            pl.BlockSpec((hdim, block_c), lambda i, j: (j, i)),
            pl.BlockSpec((1, 1, block_c), lambda i, j: (j, 0, i)),
        ],
        out_specs=pl.BlockSpec((b_sz, block_c), lambda i, j: (0, i)),
        out_shape=jax.ShapeDtypeStruct((b_sz, cdim), jnp.float32),
    )(h, gw, gb.reshape(1, n_experts), ebias.reshape(1, n_experts),
      ew.reshape(n_experts * hdim, cdim), eb.reshape(n_experts, 1, cdim))


def _moe_manual_body(h_ref, gw_ref, gb_ref, ebias_ref, eb_ref, ew_hbm,
                     o_ref, buf, sem):
    hdim = h_ref.shape[1]
    n_experts = gw_ref.shape[1]
    hb = h_ref[...].astype(jnp.bfloat16)
    scores = jnp.dot(hb, gw_ref[...].astype(jnp.bfloat16),
                     preferred_element_type=jnp.float32)
    scores = (scores + gb_ref[...]) + ebias_ref[...]
    chosen = jnp.argmax(scores, axis=1)

    def copy(e):
        return pltpu.make_async_copy(
            ew_hbm.at[pl.ds(e * hdim, hdim), :], buf.at[e % 2], sem.at[e % 2])

    copy(0).start()
    for e in range(n_experts):
        if e + 1 < n_experts:
            copy(e + 1).start()
        copy(e).wait()
        mask = (chosen == e)[:, None]
        hm = jnp.where(mask, hb, jnp.bfloat16(0.0))
        contrib = jnp.dot(hm, buf[e % 2].astype(jnp.bfloat16),
                          preferred_element_type=jnp.float32)
        contrib = contrib + jnp.where(mask, eb_ref[e][None, :], 0.0)
        if e == 0:
            o_ref[...] = contrib
        else:
            o_ref[...] += contrib


def _moe_head_manual(h, gw, gb, ebias, ew, eb):
    b_sz, hdim = h.shape
    n_experts, _, cdim = ew.shape
    return pl.pallas_call(
        _moe_manual_body,
        in_specs=[
            pl.BlockSpec(memory_space=pltpu.VMEM),
            pl.BlockSpec(memory_space=pltpu.VMEM),
            pl.BlockSpec(memory_space=pltpu.VMEM),
            pl.BlockSpec(memory_space=pltpu.VMEM),
            pl.BlockSpec(memory_space=pltpu.VMEM),
            pl.BlockSpec(memory_space=pl.ANY),
        ],
        out_specs=pl.BlockSpec(memory_space=pltpu.VMEM),
        out_shape=jax.ShapeDtypeStruct((b_sz, cdim), jnp.float32),
        scratch_shapes=[
            pltpu.VMEM((2, hdim, cdim), jnp.float32),
            pltpu.SemaphoreType.DMA((2,)),
        ],
    )(h, gw, gb.reshape(1, n_experts), ebias.reshape(1, n_experts),
      eb, ew.reshape(n_experts * hdim, cdim))


def _conv(x, w, b, stride, pad):
    out = jax.lax.conv_general_dilated(
        x, w, (stride, stride), [(pad, pad), (pad, pad)],
        dimension_numbers=('NCHW', 'OIHW', 'NCHW'))
    return out + b[None, :, None, None]


def _bn(x, g, b):
    scale = g / jnp.sqrt(1.0 + EPS)
    return x * scale[None, :, None, None] + b[None, :, None, None]


def _maxpool(x):
    return jax.lax.reduce_window(x, -jnp.inf, jax.lax.max,
                                 (1, 1, 3, 3), (1, 1, 2, 2), 'VALID')


def kernel(x, params, expert_bias):
    p = params
    t = _maxpool(jax.nn.relu(_bn(_conv(x, p['c1w'], p['c1b'], 4, 0), p['bn1g'], p['bn1b'])))
    t = _maxpool(jax.nn.relu(_bn(_conv(t, p['c2w'], p['c2b'], 1, 2), p['bn2g'], p['bn2b'])))
    t = jax.nn.relu(_bn(_conv(t, p['c3w'], p['c3b'], 1, 1), p['bn3g'], p['bn3b']))
    t = jax.nn.relu(_bn(_conv(t, p['c4w'], p['c4b'], 1, 1), p['bn4g'], p['bn4b']))
    t = _maxpool(jax.nn.relu(_bn(_conv(t, p['c5w'], p['c5b'], 1, 1), p['bn5g'], p['bn5b'])))
    h0 = t.reshape(t.shape[0], -1)

    h1 = _fc_relu(h0, p['fc1w'], p['fc1b'], block_n=512)
    h2 = _fc_relu(h1, p['fc2w'], p['fc2b'], block_n=512)

    out = _moe_head_manual(h2, p['gw'], p['gb'], expert_bias, p['ew'], p['eb'])
    return out


# speedup vs baseline: 1.1217x; 1.0031x over previous
import jax
import jax.numpy as jnp
from jax.experimental import pallas as pl
from jax.experimental.pallas import tpu as pltpu

EPS = 1e-5


def _fc_relu_body(x_ref, w_ref, b_ref, o_ref):
    x = x_ref[...].astype(jnp.bfloat16)
    w = w_ref[...].astype(jnp.bfloat16)
    acc = jnp.dot(x, w, preferred_element_type=jnp.float32)
    o_ref[...] = jnp.maximum(acc + b_ref[...], 0.0)


def _fc_relu(x, w, b, block_n):
    m, k = x.shape
    _, n = w.shape
    grid = (n // block_n,)
    return pl.pallas_call(
        _fc_relu_body,
        grid=grid,
        in_specs=[
            pl.BlockSpec((m, k), lambda i: (0, 0)),
            pl.BlockSpec((k, block_n), lambda i: (0, i)),
            pl.BlockSpec((1, block_n), lambda i: (0, i)),
        ],
        out_specs=pl.BlockSpec((m, block_n), lambda i: (0, i)),
        out_shape=jax.ShapeDtypeStruct((m, n), jnp.float32),
    )(x, w, b.reshape(1, n))


def _moe_body(h_ref, gw_ref, gb_ref, ebias_ref, ew_ref, eb_ref, o_ref):
    e = pl.program_id(1)
    hb = h_ref[...].astype(jnp.bfloat16)
    scores = jnp.dot(hb, gw_ref[...].astype(jnp.bfloat16),
                     preferred_element_type=jnp.float32)
    scores = (scores + gb_ref[...]) + ebias_ref[...]
    chosen = jnp.argmax(scores, axis=1)
    mask = (chosen == e)[:, None]
    hm = jnp.where(mask, hb, jnp.bfloat16(0.0))
    contrib = jnp.dot(hm, ew_ref[...].astype(jnp.bfloat16),
                      preferred_element_type=jnp.float32)
    contrib = contrib + jnp.where(mask, eb_ref[0], 0.0)

    @pl.when(e == 0)
    def _init():
        o_ref[...] = contrib

    @pl.when(e > 0)
    def _acc():
        o_ref[...] += contrib


def _moe_head(h, gw, gb, ebias, ew, eb, block_c):
    b_sz, hdim = h.shape
    n_experts, _, cdim = ew.shape
    grid = (pl.cdiv(cdim, block_c), n_experts)
    return pl.pallas_call(
        _moe_body,
        grid=grid,
        in_specs=[
            pl.BlockSpec((b_sz, hdim), lambda i, j: (0, 0)),
            pl.BlockSpec((hdim, n_experts), lambda i, j: (0, 0)),
            pl.BlockSpec((1, n_experts), lambda i, j: (0, 0)),
            pl.BlockSpec((1, n_experts), lambda i, j: (0, 0)),
            pl.BlockSpec((hdim, block_c), lambda i, j: (j, i)),
            pl.BlockSpec((1, 1, block_c), lambda i, j: (j, 0, i)),
        ],
        out_specs=pl.BlockSpec((b_sz, block_c), lambda i, j: (0, i)),
        out_shape=jax.ShapeDtypeStruct((b_sz, cdim), jnp.float32),
    )(h, gw, gb.reshape(1, n_experts), ebias.reshape(1, n_experts),
      ew.reshape(n_experts * hdim, cdim), eb.reshape(n_experts, 1, cdim))


def _moe_manual_body(h_ref, gw_ref, gb_ref, ebias_ref, eb_ref, ew_hbm,
                     o_ref, buf, sem):
    hdim = h_ref.shape[1]
    n_experts = gw_ref.shape[1]
    hb = h_ref[...].astype(jnp.bfloat16)
    scores = jnp.dot(hb, gw_ref[...].astype(jnp.bfloat16),
                     preferred_element_type=jnp.float32)
    scores = (scores + gb_ref[...]) + ebias_ref[...]
    chosen = jnp.argmax(scores, axis=1)

    n_split = 4
    rows = hdim // n_split

    def copies(e):
        return [pltpu.make_async_copy(
            ew_hbm.at[pl.ds(e * hdim + s * rows, rows), :],
            buf.at[e % 2, pl.ds(s * rows, rows), :],
            sem.at[e % 2]) for s in range(n_split)]

    def start(e):
        for c in copies(e):
            c.start()

    def wait(e):
        for c in copies(e):
            c.wait()

    start(0)
    for e in range(n_experts):
        if e + 1 < n_experts:
            start(e + 1)
        wait(e)
        mask = (chosen == e)[:, None]
        hm = jnp.where(mask, hb, jnp.bfloat16(0.0))
        contrib = jnp.dot(hm, buf[e % 2].astype(jnp.bfloat16),
                          preferred_element_type=jnp.float32)
        contrib = contrib + jnp.where(mask, eb_ref[e][None, :], 0.0)
        if e == 0:
            o_ref[...] = contrib
        else:
            o_ref[...] += contrib


def _moe_head_manual(h, gw, gb, ebias, ew, eb):
    b_sz, hdim = h.shape
    n_experts, _, cdim = ew.shape
    return pl.pallas_call(
        _moe_manual_body,
        in_specs=[
            pl.BlockSpec(memory_space=pltpu.VMEM),
            pl.BlockSpec(memory_space=pltpu.VMEM),
            pl.BlockSpec(memory_space=pltpu.VMEM),
            pl.BlockSpec(memory_space=pltpu.VMEM),
            pl.BlockSpec(memory_space=pltpu.VMEM),
            pl.BlockSpec(memory_space=pl.ANY),
        ],
        out_specs=pl.BlockSpec(memory_space=pltpu.VMEM),
        out_shape=jax.ShapeDtypeStruct((b_sz, cdim), jnp.float32),
        scratch_shapes=[
            pltpu.VMEM((2, hdim, cdim), jnp.float32),
            pltpu.SemaphoreType.DMA((2,)),
        ],
    )(h, gw, gb.reshape(1, n_experts), ebias.reshape(1, n_experts),
      eb, ew.reshape(n_experts * hdim, cdim))


def _conv(x, w, b, stride, pad):
    out = jax.lax.conv_general_dilated(
        x, w, (stride, stride), [(pad, pad), (pad, pad)],
        dimension_numbers=('NCHW', 'OIHW', 'NCHW'))
    return out + b[None, :, None, None]


def _bn(x, g, b):
    scale = g / jnp.sqrt(1.0 + EPS)
    return x * scale[None, :, None, None] + b[None, :, None, None]


def _maxpool(x):
    return jax.lax.reduce_window(x, -jnp.inf, jax.lax.max,
                                 (1, 1, 3, 3), (1, 1, 2, 2), 'VALID')


def kernel(x, params, expert_bias):
    p = params
    t = _maxpool(jax.nn.relu(_bn(_conv(x, p['c1w'], p['c1b'], 4, 0), p['bn1g'], p['bn1b'])))
    t = _maxpool(jax.nn.relu(_bn(_conv(t, p['c2w'], p['c2b'], 1, 2), p['bn2g'], p['bn2b'])))
    t = jax.nn.relu(_bn(_conv(t, p['c3w'], p['c3b'], 1, 1), p['bn3g'], p['bn3b']))
    t = jax.nn.relu(_bn(_conv(t, p['c4w'], p['c4b'], 1, 1), p['bn4g'], p['bn4b']))
    t = _maxpool(jax.nn.relu(_bn(_conv(t, p['c5w'], p['c5b'], 1, 1), p['bn5g'], p['bn5b'])))
    h0 = t.reshape(t.shape[0], -1)

    h1 = _fc_relu(h0, p['fc1w'], p['fc1b'], block_n=512)
    h2 = _fc_relu(h1, p['fc2w'], p['fc2b'], block_n=512)

    out = _moe_head_manual(h2, p['gw'], p['gb'], expert_bias, p['ew'], p['eb'])
    return out
